# Initial kernel scaffold; baseline (speedup 1.0000x reference)
import functools

import jax
import jax.numpy as jnp
from jax import lax
from jax.experimental import pallas as pl
from jax.experimental.pallas import tpu as pltpu
from jax.experimental.pallas import tpu_sc as plsc

ETA = 10
NC = 2
NS = 16
NW = NC * NS
L = 16
IC = 128


def _body(s_hbm, p_hbm, o_hbm, rand_hbm, mask_hbm, uniq_hbm, ent_hbm, rel_hbm,
          inp_out, corr_out,
          sidx, pidx, oidx, rand_b, mask_b, repl_b,
          es, ep, eo, wbuf, rbuf, score, sem):
    B = s_hbm.shape[0] * s_hbm.shape[1]
    CH = B // NW
    G = CH // L
    NI = CH // IC
    K = ent_hbm.shape[1]

    cid = lax.axis_index("c")
    sid = lax.axis_index("s")
    wid = sid * NC + cid
    base = pl.multiple_of(wid * CH, CH)
    irow = wid * NI

    iot = lax.iota(jnp.int32, L)

    pltpu.sync_copy(s_hbm.at[pl.ds(irow, NI), :], sidx)
    pltpu.sync_copy(p_hbm.at[pl.ds(irow, NI), :], pidx)
    pltpu.sync_copy(o_hbm.at[pl.ds(irow, NI), :], oidx)

    cps = []
    for i in range(NI):
        cps.append(pltpu.async_copy(ent_hbm.at[sidx.at[i]],
                                    es.at[pl.ds(i * IC, IC), :], sem))
        cps.append(pltpu.async_copy(rel_hbm.at[pidx.at[i]],
                                    ep.at[pl.ds(i * IC, IC), :], sem))
        cps.append(pltpu.async_copy(ent_hbm.at[oidx.at[i]],
                                    eo.at[pl.ds(i * IC, IC), :], sem))
    for cp in cps:
        cp.wait()

    def inp_group(g, _):
        rows = g * L + iot
        acc = jnp.zeros((L,), jnp.float32)
        for k in range(K):
            kv = jnp.full((L,), k, jnp.int32)
            es_c = plsc.load_gather(es, [rows, kv])
            ep_c = plsc.load_gather(ep, [rows, kv])
            eo_c = plsc.load_gather(eo, [rows, kv])
            sp = es_c * ep_c
            po = ep_c * eo_c
            acc = acc + sp * eo_c
            plsc.store_scatter(wbuf, [rows, kv], sp)
            plsc.store_scatter(wbuf, [rows + CH, kv], po)
        score[pl.ds(g * L, L)] = acc
        return 0

    lax.fori_loop(0, G, inp_group, 0)
    pltpu.sync_copy(score, inp_out.at[pl.ds(base, CH)])

    def corr_block(c, _):
        brow = pl.multiple_of(c * (B // IC) + irow, NI)
        pltpu.sync_copy(rand_hbm.at[pl.ds(brow, NI), :], rand_b)
        pltpu.sync_copy(mask_hbm.at[pl.ds(brow, NI), :], mask_b)
        rcps = [pltpu.async_copy(uniq_hbm.at[rand_b.at[i]], repl_b.at[i], sem)
                for i in range(NI)]
        for cp in rcps:
            cp.wait()
        ecps = [pltpu.async_copy(ent_hbm.at[repl_b.at[i]],
                                 rbuf.at[pl.ds(i * IC, IC), :], sem)
                for i in range(NI)]
        for cp in ecps:
            cp.wait()

        def corr_group(g, _):
            rows = g * L + iot
            mrow = g // (IC // L)
            mcol = (g % (IC // L)) * L
            m = plsc.load_gather(mask_b, [jnp.full((L,), 0, jnp.int32) + mrow,
                                          mcol + iot])
            sel = rows + jnp.where(m == 1, 0, CH)
            acc = jnp.zeros((L,), jnp.float32)
            for k in range(K):
                kv = jnp.full((L,), k, jnp.int32)
                r_c = plsc.load_gather(rbuf, [rows, kv])
                w_c = plsc.load_gather(wbuf, [sel, kv])
                acc = acc + r_c * w_c
            score[pl.ds(g * L, L)] = acc
            return 0

        lax.fori_loop(0, G, corr_group, 0)
        off = pl.multiple_of(c * B + base, CH)
        pltpu.sync_copy(score, corr_out.at[pl.ds(off, CH)])
        return 0

    lax.fori_loop(0, ETA, corr_block, 0)


def kernel(inputs, unique_entities, ent_emb, rel_emb):
    B = inputs.shape[0]
    n = B * ETA
    k1, k2 = jax.random.split(jax.random.key(42))
    rand_idx = jax.random.randint(k1, (n,), 0, unique_entities.shape[0])
    keep_mask = jax.random.randint(k2, (n,), 0, 2)

    s_col = inputs[:, 0].astype(jnp.int32).reshape(-1, IC)
    p_col = inputs[:, 1].astype(jnp.int32).reshape(-1, IC)
    o_col = inputs[:, 2].astype(jnp.int32).reshape(-1, IC)
    rand2d = rand_idx.astype(jnp.int32).reshape(-1, IC)
    mask2d = keep_mask.astype(jnp.int32).reshape(-1, IC)
    uniq = unique_entities.astype(jnp.int32)

    CH = B // NW
    NI = CH // IC
    K = ent_emb.shape[1]

    grid_kernel = pl.kernel(
        _body,
        out_type=(jax.ShapeDtypeStruct((B,), jnp.float32),
                  jax.ShapeDtypeStruct((n,), jnp.float32)),
        mesh=plsc.VectorSubcoreMesh(core_axis_name="c", subcore_axis_name="s",
                                    num_cores=NC, num_subcores=NS),
        scratch_types=[
            pltpu.VMEM((NI, IC), jnp.int32),
            pltpu.VMEM((NI, IC), jnp.int32),
            pltpu.VMEM((NI, IC), jnp.int32),
            pltpu.VMEM((NI, IC), jnp.int32),
            pltpu.VMEM((NI, IC), jnp.int32),
            pltpu.VMEM((NI, IC), jnp.int32),
            pltpu.VMEM((CH, K), jnp.float32),
            pltpu.VMEM((CH, K), jnp.float32),
            pltpu.VMEM((CH, K), jnp.float32),
            pltpu.VMEM((2 * CH, K), jnp.float32),
            pltpu.VMEM((CH, K), jnp.float32),
            pltpu.VMEM((CH,), jnp.float32),
            pltpu.SemaphoreType.DMA,
        ],
    )
    inp_score, corr_score = grid_kernel(s_col, p_col, o_col, rand2d, mask2d,
                                        uniq, ent_emb, rel_emb)
    return (inp_score, corr_score)


# speedup vs baseline: 1.5231x; 1.5231x over previous
import jax
import jax.numpy as jnp
from jax import lax
from jax.experimental import pallas as pl
from jax.experimental.pallas import tpu as pltpu
from jax.experimental.pallas import tpu_sc as plsc

ETA = 10
NC = 2
NS = 16
NW = NC * NS
L = 16
SUB = 128

_DN = lax.GatherDimensionNumbers(offset_dims=(), collapsed_slice_dims=(0,),
                                 start_index_map=(0,))

def _lane_mask(iota, jj):
    xf = jnp.bitwise_xor(iota, jj).astype(jnp.float32)
    return 1.0 - jnp.minimum(xf, 1.0)


def _bcast(v, jj):
    idx = jnp.full((L,), jj, jnp.int32)
    return lax.gather(v, idx[:, None], _DN, (1,),
                      mode=lax.GatherScatterMode.PROMISE_IN_BOUNDS)


def _hsum(v, iota):
    for sh in (8, 4, 2, 1):
        perm = iota ^ sh
        v = v + lax.gather(v, perm[:, None], _DN, (1,),
                           mode=lax.GatherScatterMode.PROMISE_IN_BOUNDS)
    return v


def _blend(a, b, t):
    return a + (b - a) * t


def _subrow(bbuf, row, b0f, b1f):
    h = [bbuf[row, pl.ds(i * L, L)] for i in range(8)]
    lo = _blend(_blend(h[0], h[2], b0f), _blend(h[4], h[6], b0f), b1f)
    hi = _blend(_blend(h[1], h[3], b0f), _blend(h[5], h[7], b0f), b1f)
    return lo, hi


def _qbits(v, jj):
    b = _bcast(v, jj)
    b0f = jnp.bitwise_and(b, 1).astype(jnp.float32)
    b1f = jnp.bitwise_and(lax.shift_right_logical(b, 1), 1).astype(jnp.float32)
    return b0f, b1f


def _body(s_hbm, p_hbm, o_hbm, rand_hbm, mask_hbm, uniq_hbm, ent_hbm, rel_hbm,
          inp_out, corr_out,
          ids_s, ids_p, ids_o, ids_r, ids_m, ids_e, blki,
          bb_s, bb_p, bb_o, w1, w2, score, sem):
    B = s_hbm.shape[0] * s_hbm.shape[1]
    CH = B // NW
    NSUB = CH // SUB
    NG = SUB // L

    cid = lax.axis_index("c")
    sid = lax.axis_index("s")
    wid = sid * NC + cid
    base = wid * CH
    irow = wid * NSUB

    iota = lax.iota(jnp.int32, L)

    pltpu.sync_copy(s_hbm.at[pl.ds(irow, NSUB), :], ids_s)
    pltpu.sync_copy(p_hbm.at[pl.ds(irow, NSUB), :], ids_p)
    pltpu.sync_copy(o_hbm.at[pl.ds(irow, NSUB), :], ids_o)

    def store_blockids(src, sub, dst_row):
        for t in range(NG):
            ev = src[sub, pl.ds(t * L, L)]
            blki[dst_row, pl.ds(t * L, L)] = lax.shift_right_logical(ev, 2)

    for sub in range(NSUB):
        store_blockids(ids_s, sub, 0)
        store_blockids(ids_p, sub, 1)
        store_blockids(ids_o, sub, 2)
        cps = [pltpu.async_copy(ent_hbm.at[blki.at[0]], bb_s, sem),
               pltpu.async_copy(rel_hbm.at[blki.at[1]], bb_p, sem),
               pltpu.async_copy(ent_hbm.at[blki.at[2]], bb_o, sem)]
        for cp in cps:
            cp.wait()

        def orig_group(g, _):
            off = g * L
            sv = ids_s[sub, pl.ds(off, L)]
            pv = ids_p[sub, pl.ds(off, L)]
            ov = ids_o[sub, pl.ds(off, L)]
            accv = jnp.zeros((L,), jnp.float32)
            for jj in range(L):
                row = off + jj
                s0, s1 = _subrow(bb_s, row, *_qbits(sv, jj))
                p0, p1 = _subrow(bb_p, row, *_qbits(pv, jj))
                o0, o1 = _subrow(bb_o, row, *_qbits(ov, jj))
                sp0 = s0 * p0
                sp1 = s1 * p1
                po0 = p0 * o0
                po1 = p1 * o1
                wr = sub * (SUB // 4) + g * 4 + (jj >> 2)
                wc = (jj & 3) * 32
                w1[wr, pl.ds(wc, L)] = sp0
                w1[wr, pl.ds(wc + L, L)] = sp1
                w2[wr, pl.ds(wc, L)] = po0
                w2[wr, pl.ds(wc + L, L)] = po1
                tot = _hsum(sp0 * o0 + sp1 * o1, iota)
                accv = accv + tot * _lane_mask(iota, jj)
            score[pl.ds(off, L)] = accv
            return 0

        lax.fori_loop(0, NG, orig_group, 0)
        pltpu.sync_copy(score, inp_out.at[pl.ds(base + sub * SUB, SUB)])

    def corr_block(c, _):
        brow = c * (B // SUB) + irow
        pltpu.sync_copy(rand_hbm.at[pl.ds(brow, NSUB), :], ids_r)
        pltpu.sync_copy(mask_hbm.at[pl.ds(brow, NSUB), :], ids_m)

        def corr_sub(sub, _):
            pltpu.async_copy(uniq_hbm.at[ids_r.at[sub]], ids_e.at[sub],
                             sem).wait()
            for t in range(NG):
                ev = ids_e[sub, pl.ds(t * L, L)]
                blki[3, pl.ds(t * L, L)] = lax.shift_right_logical(ev, 2)
            pltpu.async_copy(ent_hbm.at[blki.at[3]], bb_s, sem).wait()

            def corr_group(g, _):
                off = g * L
                ev = ids_e[sub, pl.ds(off, L)]
                mv = ids_m[sub, pl.ds(off, L)]
                accv = jnp.zeros((L,), jnp.float32)
                for jj in range(L):
                    row = off + jj
                    r0, r1 = _subrow(bb_s, row, *_qbits(ev, jj))
                    mf = _bcast(mv, jj).astype(jnp.float32)
                    wr = sub * (SUB // 4) + g * 4 + (jj >> 2)
                    wc = (jj & 3) * 32
                    w1a = w1[wr, pl.ds(wc, L)]
                    w1b = w1[wr, pl.ds(wc + L, L)]
                    w2a = w2[wr, pl.ds(wc, L)]
                    w2b = w2[wr, pl.ds(wc + L, L)]
                    wa = _blend(w2a, w1a, mf)
                    wb = _blend(w2b, w1b, mf)
                    tot = _hsum(r0 * wa + r1 * wb, iota)
                    accv = accv + tot * _lane_mask(iota, jj)
                score[pl.ds(off, L)] = accv
                return 0

            lax.fori_loop(0, NG, corr_group, 0)
            off_out = c * B + base + sub * SUB
            pltpu.sync_copy(score, corr_out.at[pl.ds(off_out, SUB)])
            return 0

        lax.fori_loop(0, NSUB, corr_sub, 0)
        return 0

    lax.fori_loop(0, ETA, corr_block, 0)


def kernel(inputs, unique_entities, ent_emb, rel_emb):
    B = inputs.shape[0]
    n = B * ETA
    k1, k2 = jax.random.split(jax.random.key(42))
    rand_idx = jax.random.randint(k1, (n,), 0, unique_entities.shape[0])
    keep_mask = jax.random.randint(k2, (n,), 0, 2)

    s2d = inputs[:, 0].astype(jnp.int32).reshape(-1, SUB)
    p2d = inputs[:, 1].astype(jnp.int32).reshape(-1, SUB)
    o2d = inputs[:, 2].astype(jnp.int32).reshape(-1, SUB)
    rand2d = rand_idx.astype(jnp.int32).reshape(-1, SUB)
    mask2d = keep_mask.astype(jnp.int32).reshape(-1, SUB)
    uniq = unique_entities.astype(jnp.int32)
    ent128 = ent_emb.reshape(-1, 4 * ent_emb.shape[1])
    rel128 = rel_emb.reshape(-1, 4 * rel_emb.shape[1])

    CH = B // NW
    K = ent_emb.shape[1]

    grid_kernel = pl.kernel(
        _body,
        out_type=(jax.ShapeDtypeStruct((B,), jnp.float32),
                  jax.ShapeDtypeStruct((n,), jnp.float32)),
        mesh=plsc.VectorSubcoreMesh(core_axis_name="c", subcore_axis_name="s",
                                    num_cores=NC, num_subcores=NS),
        scratch_types=[
            pltpu.VMEM((CH // SUB, SUB), jnp.int32),
            pltpu.VMEM((CH // SUB, SUB), jnp.int32),
            pltpu.VMEM((CH // SUB, SUB), jnp.int32),
            pltpu.VMEM((CH // SUB, SUB), jnp.int32),
            pltpu.VMEM((CH // SUB, SUB), jnp.int32),
            pltpu.VMEM((CH // SUB, SUB), jnp.int32),
            pltpu.VMEM((4, SUB), jnp.int32),
            pltpu.VMEM((SUB, 4 * K), jnp.float32),
            pltpu.VMEM((SUB, 4 * K), jnp.float32),
            pltpu.VMEM((SUB, 4 * K), jnp.float32),
            pltpu.VMEM((CH // 4, 4 * K), jnp.float32),
            pltpu.VMEM((CH // 4, 4 * K), jnp.float32),
            pltpu.VMEM((SUB,), jnp.float32),
            pltpu.SemaphoreType.DMA,
        ],
    )
    inp_score, corr_score = grid_kernel(s2d, p2d, o2d, rand2d, mask2d,
                                        uniq, ent128, rel128)
    return (inp_score, corr_score)


# speedup vs baseline: 1.7353x; 1.1394x over previous
import jax
import jax.numpy as jnp
from jax import lax
from jax.experimental import pallas as pl
from jax.experimental.pallas import tpu as pltpu
from jax.experimental.pallas import tpu_sc as plsc

ETA = 10
NC = 2
NS = 16
NW = NC * NS
L = 16
SUB = 128

_DN = lax.GatherDimensionNumbers(offset_dims=(), collapsed_slice_dims=(0,),
                                 start_index_map=(0,))


def _bcast(v, jj):
    idx = jnp.full((L,), jj, jnp.int32)
    return lax.gather(v, idx[:, None], _DN, (1,),
                      mode=lax.GatherScatterMode.PROMISE_IN_BOUNDS)


def _hsum(v, iota):
    for sh in (8, 4, 2, 1):
        perm = iota ^ sh
        v = v + lax.gather(v, perm[:, None], _DN, (1,),
                           mode=lax.GatherScatterMode.PROMISE_IN_BOUNDS)
    return v


def _lane_mask(iota, jj):
    xf = jnp.bitwise_xor(iota, jj).astype(jnp.float32)
    return 1.0 - jnp.minimum(xf, 1.0)


def _blend(a, b, t):
    return a + (b - a) * t


def _subrow(bbuf, row, b0f, b1f):
    h = [bbuf[row, pl.ds(i * L, L)] for i in range(8)]
    lo = _blend(_blend(h[0], h[2], b0f), _blend(h[4], h[6], b0f), b1f)
    hi = _blend(_blend(h[1], h[3], b0f), _blend(h[5], h[7], b0f), b1f)
    return lo, hi


def _qbits(v, jj):
    b = _bcast(v, jj)
    b0f = jnp.bitwise_and(b, 1).astype(jnp.float32)
    b1f = jnp.bitwise_and(lax.shift_right_logical(b, 1), 1).astype(jnp.float32)
    return b0f, b1f


def _body(s_hbm, p_hbm, o_hbm, rand_hbm, mask_hbm, uniq_hbm, ent_hbm, rel_hbm,
          inp_out, corr_out,
          ids_s, ids_p, ids_o, ids_r, ids_m, ids_e, blka,
          bb_s, bb_p, bb_o, w1, w2, score, sem, sem2):
    B = s_hbm.shape[0] * s_hbm.shape[1]
    CH = B // NW
    NSUB = CH // SUB
    NG = SUB // L
    NBLK = ETA * NSUB

    cid = lax.axis_index("c")
    sid = lax.axis_index("s")
    wid = sid * NC + cid
    base = wid * CH
    irow = wid * NSUB

    iota = lax.iota(jnp.int32, L)

    pltpu.sync_copy(s_hbm.at[pl.ds(irow, NSUB), :], ids_s)
    pltpu.sync_copy(p_hbm.at[pl.ds(irow, NSUB), :], ids_p)
    pltpu.sync_copy(o_hbm.at[pl.ds(irow, NSUB), :], ids_o)
    pltpu.sync_copy(rand_hbm.at[pl.ds(wid * NBLK, NBLK), :], ids_r)
    pltpu.sync_copy(mask_hbm.at[pl.ds(wid * NBLK, NBLK), :], ids_m)

    ucps = [pltpu.async_copy(uniq_hbm.at[ids_r.at[r]], ids_e.at[r], sem2)
            for r in range(NBLK)]

    bbs = (bb_s, bb_o)
    for sub in range(NSUB):
        for t in range(NG):
            blka[0, pl.ds(t * L, L)] = lax.shift_right_logical(
                ids_s[sub, pl.ds(t * L, L)], 2)
            blka[1, pl.ds(t * L, L)] = lax.shift_right_logical(
                ids_p[sub, pl.ds(t * L, L)], 2)
            blka[2, pl.ds(t * L, L)] = lax.shift_right_logical(
                ids_o[sub, pl.ds(t * L, L)], 2)
        cps = [pltpu.async_copy(ent_hbm.at[blka.at[0]], bb_s, sem),
               pltpu.async_copy(rel_hbm.at[blka.at[1]], bb_p, sem),
               pltpu.async_copy(ent_hbm.at[blka.at[2]], bb_o, sem)]
        for cp in cps:
            cp.wait()

        def orig_group(g, _, sub=sub):
            off = g * L
            sv = ids_s[sub, pl.ds(off, L)]
            pv = ids_p[sub, pl.ds(off, L)]
            ov = ids_o[sub, pl.ds(off, L)]
            accv = jnp.zeros((L,), jnp.float32)
            for jj in range(L):
                row = off + jj
                s0, s1 = _subrow(bb_s, row, *_qbits(sv, jj))
                p0, p1 = _subrow(bb_p, row, *_qbits(pv, jj))
                o0, o1 = _subrow(bb_o, row, *_qbits(ov, jj))
                sp0 = s0 * p0
                sp1 = s1 * p1
                po0 = p0 * o0
                po1 = p1 * o1
                wr = sub * (SUB // 4) + g * 4 + (jj >> 2)
                wc = (jj & 3) * 32
                w1[wr, pl.ds(wc, L)] = sp0
                w1[wr, pl.ds(wc + L, L)] = sp1
                w2[wr, pl.ds(wc, L)] = po0
                w2[wr, pl.ds(wc + L, L)] = po1
                tot = _hsum(sp0 * o0 + sp1 * o1, iota)
                accv = accv + tot * _lane_mask(iota, jj)
            score[pl.ds(off, L)] = accv
            return 0

        lax.fori_loop(0, NG, orig_group, 0)
        pltpu.sync_copy(score, inp_out.at[pl.ds(base + sub * SUB, SUB)])

    for cp in ucps:
        cp.wait()

    def mk_blk(r, _):
        for t in range(NG):
            blka[r, pl.ds(t * L, L)] = lax.shift_right_logical(
                ids_e[r, pl.ds(t * L, L)], 2)
        return 0

    lax.fori_loop(0, NBLK, mk_blk, 0)

    pltpu.async_copy(ent_hbm.at[blka.at[0]], bbs[0], sem)

    def corr_block(c, _):
        for sub in range(NSUB):
            m = c * NSUB + sub
            bb = bbs[sub % 2]
            nbb = bbs[(sub + 1) % 2]
            pltpu.make_async_copy(ent_hbm.at[blka.at[m]], bb, sem).wait()

            @pl.when(m + 1 < NBLK)
            def _():
                pltpu.async_copy(ent_hbm.at[blka.at[m + 1]], nbb, sem)

            def corr_group(g, _, sub=sub, bb=bb):
                off = g * L
                r = c * NSUB + sub
                ev = ids_e[r, pl.ds(off, L)]
                mv = ids_m[r, pl.ds(off, L)]
                accv = jnp.zeros((L,), jnp.float32)
                for jj in range(L):
                    row = off + jj
                    r0, r1 = _subrow(bb, row, *_qbits(ev, jj))
                    mf = _bcast(mv, jj).astype(jnp.float32)
                    wr = sub * (SUB // 4) + g * 4 + (jj >> 2)
                    wc = (jj & 3) * 32
                    w1a = w1[wr, pl.ds(wc, L)]
                    w1b = w1[wr, pl.ds(wc + L, L)]
                    w2a = w2[wr, pl.ds(wc, L)]
                    w2b = w2[wr, pl.ds(wc + L, L)]
                    wa = _blend(w2a, w1a, mf)
                    wb = _blend(w2b, w1b, mf)
                    tot = _hsum(r0 * wa + r1 * wb, iota)
                    accv = accv + tot * _lane_mask(iota, jj)
                score[pl.ds(off, L)] = accv
                return 0

            lax.fori_loop(0, NG, corr_group, 0)
            off_out = c * B + base + sub * SUB
            pltpu.sync_copy(score, corr_out.at[pl.ds(off_out, SUB)])
        return 0

    lax.fori_loop(0, ETA, corr_block, 0)


def kernel(inputs, unique_entities, ent_emb, rel_emb):
    B = inputs.shape[0]
    n = B * ETA
    k1, k2 = jax.random.split(jax.random.key(42))
    rand_idx = jax.random.randint(k1, (n,), 0, unique_entities.shape[0])
    keep_mask = jax.random.randint(k2, (n,), 0, 2)

    CH = B // NW
    NSUB = CH // SUB

    s2d = inputs[:, 0].astype(jnp.int32).reshape(-1, SUB)
    p2d = inputs[:, 1].astype(jnp.int32).reshape(-1, SUB)
    o2d = inputs[:, 2].astype(jnp.int32).reshape(-1, SUB)

    def worker_major(x):
        return (x.astype(jnp.int32)
                .reshape(ETA, NW, NSUB, SUB)
                .transpose(1, 0, 2, 3)
                .reshape(NW * ETA * NSUB, SUB))

    rand2d = worker_major(rand_idx)
    mask2d = worker_major(keep_mask)
    uniq = unique_entities.astype(jnp.int32)
    ent128 = ent_emb.reshape(-1, 4 * ent_emb.shape[1])
    rel128 = rel_emb.reshape(-1, 4 * rel_emb.shape[1])

    K = ent_emb.shape[1]
    NBLK = ETA * NSUB

    grid_kernel = pl.kernel(
        _body,
        out_type=(jax.ShapeDtypeStruct((B,), jnp.float32),
                  jax.ShapeDtypeStruct((n,), jnp.float32)),
        mesh=plsc.VectorSubcoreMesh(core_axis_name="c", subcore_axis_name="s",
                                    num_cores=NC, num_subcores=NS),
        scratch_types=[
            pltpu.VMEM((NSUB, SUB), jnp.int32),
            pltpu.VMEM((NSUB, SUB), jnp.int32),
            pltpu.VMEM((NSUB, SUB), jnp.int32),
            pltpu.VMEM((NBLK, SUB), jnp.int32),
            pltpu.VMEM((NBLK, SUB), jnp.int32),
            pltpu.VMEM((NBLK, SUB), jnp.int32),
            pltpu.VMEM((NBLK, SUB), jnp.int32),
            pltpu.VMEM((SUB, 4 * K), jnp.float32),
            pltpu.VMEM((SUB, 4 * K), jnp.float32),
            pltpu.VMEM((SUB, 4 * K), jnp.float32),
            pltpu.VMEM((CH // 4, 4 * K), jnp.float32),
            pltpu.VMEM((CH // 4, 4 * K), jnp.float32),
            pltpu.VMEM((SUB,), jnp.float32),
            pltpu.SemaphoreType.DMA,
            pltpu.SemaphoreType.DMA,
        ],
    )
    inp_score, corr_score = grid_kernel(s2d, p2d, o2d, rand2d, mask2d,
                                        uniq, ent128, rel128)
    return (inp_score, corr_score)


# speedup vs baseline: 1.8160x; 1.0465x over previous
import jax
import jax.numpy as jnp
from jax import lax
from jax.experimental import pallas as pl
from jax.experimental.pallas import tpu as pltpu
from jax.experimental.pallas import tpu_sc as plsc

ETA = 10
NC = 2
NS = 16
NW = NC * NS
L = 16
SUB = 128

_DN = lax.GatherDimensionNumbers(offset_dims=(), collapsed_slice_dims=(0,),
                                 start_index_map=(0,))


def _bcast(v, jj):
    idx = jnp.full((L,), jj, jnp.int32)
    return lax.gather(v, idx[:, None], _DN, (1,),
                      mode=lax.GatherScatterMode.PROMISE_IN_BOUNDS)


def _perm_xor(v, iota, s):
    perm = iota ^ s
    return lax.gather(v, perm[:, None], _DN, (1,),
                      mode=lax.GatherScatterMode.PROMISE_IN_BOUNDS)


def _tree_reduce16(vecs, iota):
    for s, bit in ((8, 3), (4, 2), (2, 1), (1, 0)):
        m = jnp.bitwise_and(lax.shift_right_logical(iota, bit), 1).astype(
            jnp.float32)
        half = len(vecs) // 2
        nxt = []
        for j in range(half):
            fa = vecs[j] + _perm_xor(vecs[j], iota, s)
            fb = vecs[j + half] + _perm_xor(vecs[j + half], iota, s)
            nxt.append(_blend(fa, fb, m))
        vecs = nxt
    return vecs[0]


def _blend(a, b, t):
    return a + (b - a) * t


def _row2(bbuf, row):
    return bbuf[row, pl.ds(0, L)], bbuf[row, pl.ds(L, L)]


def _body(s_hbm, p_hbm, o_hbm, rand_hbm, mask_hbm, uniq_hbm, ent_hbm, rel_hbm,
          inp_out, corr_out,
          ids_s, ids_p, ids_o, ids_r, ids_m, ids_e,
          bb_s, bb_p, bb_o, w1, w2, score, sem, sem2):
    B = s_hbm.shape[0] * s_hbm.shape[1]
    CH = B // NW
    NSUB = CH // SUB
    NG = SUB // L
    NBLK = ETA * NSUB

    cid = lax.axis_index("c")
    sid = lax.axis_index("s")
    wid = sid * NC + cid
    base = wid * CH
    irow = wid * NSUB

    iota = lax.iota(jnp.int32, L)

    pltpu.sync_copy(s_hbm.at[pl.ds(irow, NSUB), :], ids_s)
    pltpu.sync_copy(p_hbm.at[pl.ds(irow, NSUB), :], ids_p)
    pltpu.sync_copy(o_hbm.at[pl.ds(irow, NSUB), :], ids_o)
    pltpu.sync_copy(rand_hbm.at[pl.ds(wid * NBLK, NBLK), :], ids_r)
    pltpu.sync_copy(mask_hbm.at[pl.ds(wid * NBLK, NBLK), :], ids_m)

    ucps = [pltpu.async_copy(uniq_hbm.at[ids_r.at[r]], ids_e.at[r], sem2)
            for r in range(NBLK)]

    bbs = (bb_s, bb_o)
    for sub in range(NSUB):
        cps = [pltpu.async_copy(ent_hbm.at[ids_s.at[sub]], bb_s, sem),
               pltpu.async_copy(rel_hbm.at[ids_p.at[sub]], bb_p, sem),
               pltpu.async_copy(ent_hbm.at[ids_o.at[sub]], bb_o, sem)]
        for cp in cps:
            cp.wait()

        def orig_group(g, _, sub=sub):
            off = g * L
            prods = []
            for jj in range(L):
                row = off + jj
                s0, s1 = _row2(bb_s, row)
                p0, p1 = _row2(bb_p, row)
                o0, o1 = _row2(bb_o, row)
                sp0 = s0 * p0
                sp1 = s1 * p1
                po0 = p0 * o0
                po1 = p1 * o1
                wr = sub * (SUB // 4) + g * 4 + (jj >> 2)
                wc = (jj & 3) * 32
                w1[wr, pl.ds(wc, L)] = sp0
                w1[wr, pl.ds(wc + L, L)] = sp1
                w2[wr, pl.ds(wc, L)] = po0
                w2[wr, pl.ds(wc + L, L)] = po1
                prods.append(sp0 * o0 + sp1 * o1)
            score[pl.ds(off, L)] = _tree_reduce16(prods, iota)
            return 0

        lax.fori_loop(0, NG, orig_group, 0)
        pltpu.sync_copy(score, inp_out.at[pl.ds(base + sub * SUB, SUB)])

    for cp in ucps:
        cp.wait()

    pltpu.async_copy(ent_hbm.at[ids_e.at[0]], bbs[0], sem)

    def corr_block(c, _):
        for sub in range(NSUB):
            m = c * NSUB + sub
            bb = bbs[sub % 2]
            nbb = bbs[(sub + 1) % 2]
            pltpu.make_async_copy(ent_hbm.at[ids_e.at[m]], bb, sem).wait()

            @pl.when(m + 1 < NBLK)
            def _():
                pltpu.async_copy(ent_hbm.at[ids_e.at[m + 1]], nbb, sem)

            def corr_group(g, _, sub=sub, bb=bb):
                off = g * L
                r = c * NSUB + sub
                mv = ids_m[r, pl.ds(off, L)]
                mvf = mv.astype(jnp.float32)
                prods = []
                for jj in range(L):
                    row = off + jj
                    r0, r1 = _row2(bb, row)
                    mf = _bcast(mvf, jj)
                    wr = sub * (SUB // 4) + g * 4 + (jj >> 2)
                    wc = (jj & 3) * 32
                    w1a = w1[wr, pl.ds(wc, L)]
                    w1b = w1[wr, pl.ds(wc + L, L)]
                    w2a = w2[wr, pl.ds(wc, L)]
                    w2b = w2[wr, pl.ds(wc + L, L)]
                    wa = _blend(w2a, w1a, mf)
                    wb = _blend(w2b, w1b, mf)
                    prods.append(r0 * wa + r1 * wb)
                score[pl.ds(off, L)] = _tree_reduce16(prods, iota)
                return 0

            lax.fori_loop(0, NG, corr_group, 0)
            off_out = c * B + base + sub * SUB
            pltpu.sync_copy(score, corr_out.at[pl.ds(off_out, SUB)])
        return 0

    lax.fori_loop(0, ETA, corr_block, 0)


def kernel(inputs, unique_entities, ent_emb, rel_emb):
    B = inputs.shape[0]
    n = B * ETA
    k1, k2 = jax.random.split(jax.random.key(42))
    rand_idx = jax.random.randint(k1, (n,), 0, unique_entities.shape[0])
    keep_mask = jax.random.randint(k2, (n,), 0, 2)

    CH = B // NW
    NSUB = CH // SUB

    s2d = inputs[:, 0].astype(jnp.int32).reshape(-1, SUB)
    p2d = inputs[:, 1].astype(jnp.int32).reshape(-1, SUB)
    o2d = inputs[:, 2].astype(jnp.int32).reshape(-1, SUB)

    def worker_major(x):
        return (x.astype(jnp.int32)
                .reshape(ETA, NW, NSUB, SUB)
                .transpose(1, 0, 2, 3)
                .reshape(NW * ETA * NSUB, SUB))

    rand2d = worker_major(rand_idx)
    mask2d = worker_major(keep_mask)
    uniq = unique_entities.astype(jnp.int32)

    ent128 = jnp.pad(ent_emb, ((0, 0), (0, 128 - ent_emb.shape[1])))
    rel128 = jnp.pad(rel_emb, ((0, 0), (0, 128 - rel_emb.shape[1])))

    K = ent_emb.shape[1]
    NBLK = ETA * NSUB

    grid_kernel = pl.kernel(
        _body,
        out_type=(jax.ShapeDtypeStruct((B,), jnp.float32),
                  jax.ShapeDtypeStruct((n,), jnp.float32)),
        mesh=plsc.VectorSubcoreMesh(core_axis_name="c", subcore_axis_name="s",
                                    num_cores=NC, num_subcores=NS),
        scratch_types=[
            pltpu.VMEM((NSUB, SUB), jnp.int32),
            pltpu.VMEM((NSUB, SUB), jnp.int32),
            pltpu.VMEM((NSUB, SUB), jnp.int32),
            pltpu.VMEM((NBLK, SUB), jnp.int32),
            pltpu.VMEM((NBLK, SUB), jnp.int32),
            pltpu.VMEM((NBLK, SUB), jnp.int32),
            pltpu.VMEM((SUB, 4 * K), jnp.float32),
            pltpu.VMEM((SUB, 4 * K), jnp.float32),
            pltpu.VMEM((SUB, 4 * K), jnp.float32),
            pltpu.VMEM((CH // 4, 4 * K), jnp.float32),
            pltpu.VMEM((CH // 4, 4 * K), jnp.float32),
            pltpu.VMEM((SUB,), jnp.float32),
            pltpu.SemaphoreType.DMA,
            pltpu.SemaphoreType.DMA,
        ],
    )
    inp_score, corr_score = grid_kernel(s2d, p2d, o2d, rand2d, mask2d,
                                        uniq, ent128, rel128)
    return (inp_score, corr_score)
